# baseline (speedup 1.0000x reference)
import jax
import jax.numpy as jnp
from jax import lax
from jax.experimental import pallas as pl
from jax.experimental.pallas import tpu as pltpu
from jax.experimental.pallas import tpu_sc as plsc

_NC = 2
_NS = 16
_NW = _NC * _NS

_U_SHARE = 31248
_I_SHARE = 3120
_U_CHUNK = 248
_I_CHUNK = 240
_NCU = _U_SHARE // _U_CHUNK
_NCI = _I_SHARE // _I_CHUNK
_NBUF = 4
_K = _NBUF - 2


def _body(u_hbm, it_hbm, ego_hbm, bufs, in_sem, out_sem):
    nu = u_hbm.shape[0]
    ni = it_hbm.shape[0]

    wid = lax.axis_index("s") * _NC + lax.axis_index("c")
    base_u = wid * _U_SHARE
    base_i = wid * _I_SHARE
    n = _NCU + _NCI

    def chunk(j):
        if j < _NCU:
            off = base_u + j * _U_CHUNK
            return (u_hbm.at[pl.ds(off, _U_CHUNK)],
                    ego_hbm.at[pl.ds(off, _U_CHUNK)], _U_CHUNK)
        off = base_i + (j - _NCU) * _I_CHUNK
        return (it_hbm.at[pl.ds(off, _I_CHUNK)],
                ego_hbm.at[pl.ds(nu + off, _I_CHUNK)], _I_CHUNK)

    def in_copy(j):
        s = j % _NBUF
        src, _, r = chunk(j)
        return pltpu.make_async_copy(src, bufs.at[s, pl.ds(0, r)], in_sem.at[s])

    def out_copy(j):
        s = j % _NBUF
        _, d_ego, r = chunk(j)
        return pltpu.make_async_copy(bufs.at[s, pl.ds(0, r)], d_ego, out_sem.at[s])

    for j in range(_K):
        in_copy(j).start()
    for j in range(n):
        nxt = j + _K
        if nxt < n:
            prev = nxt - _NBUF
            if prev >= 0:
                out_copy(prev).wait()
            in_copy(nxt).start()
        in_copy(j).wait()
        out_copy(j).start()
    for j in range(n - _NBUF, n):
        out_copy(j).wait()

    ur_off = _NW * _U_SHARE
    ur = nu - ur_off
    ir_off = _NW * _I_SHARE
    ir = ni - ir_off

    @pl.when(wid == 0)
    def _():
        pltpu.sync_copy(u_hbm.at[pl.ds(ur_off, ur)], bufs.at[0, pl.ds(0, ur)])
        pltpu.sync_copy(bufs.at[0, pl.ds(0, ur)], ego_hbm.at[pl.ds(ur_off, ur)])
        pltpu.sync_copy(it_hbm.at[pl.ds(ir_off, ir)], bufs.at[0, pl.ds(0, ir)])
        pltpu.sync_copy(bufs.at[0, pl.ds(0, ir)], ego_hbm.at[pl.ds(nu + ir_off, ir)])


def kernel(user_weight, item_weight):
    n_users, emb = user_weight.shape
    n_items, _ = item_weight.shape

    run = pl.kernel(
        _body,
        out_type=jax.ShapeDtypeStruct((n_users + n_items, emb), jnp.float32),
        mesh=plsc.VectorSubcoreMesh(core_axis_name="c", subcore_axis_name="s"),
        scratch_types=[
            pltpu.VMEM((_NBUF, _U_CHUNK, emb), jnp.float32),
            pltpu.SemaphoreType.DMA((_NBUF,)),
            pltpu.SemaphoreType.DMA((_NBUF,)),
        ],
    )
    ego = run(user_weight, item_weight)
    return (user_weight, item_weight, ego)
